# Initial kernel scaffold; baseline (speedup 1.0000x reference)
import jax
import jax.numpy as jnp
from jax.experimental import pallas as pl


def kernel(input, input_embed, W_ih_0, W_hh_0, b_ih_0, b_hh_0, W_ih_1, W_hh_1, b_ih_1, b_hh_1):
    raise NotImplementedError("write your pallas kernel here")


# speedup vs baseline: 3.4588x; 3.4588x over previous
import jax
import jax.numpy as jnp
from jax.experimental import pallas as pl
from jax.experimental.pallas import tpu as pltpu

B, S, E, H = 64, 32, 1024, 1024
G4 = 4 * H
SPLIT = 50

_DN = (((1,), (1,)), ((), ()))


def _proj_kernel(x_ref, w0_ref, w1_ref, b0_ref, b1_ref, m_ref, out_ref):
    x = x_ref[...]
    z0 = jax.lax.dot_general(x, w0_ref[...], _DN,
                             preferred_element_type=jnp.float32)
    z1 = jax.lax.dot_general(x, w1_ref[...], _DN,
                             preferred_element_type=jnp.float32)
    m = m_ref[...]
    out_ref[...] = m * (z0 + b0_ref[...]) + (1.0 - m) * (z1 + b1_ref[...])


def _step_kernel(z_ref, m_ref, wh0_ref, wh1_ref, out_ref, hT_ref, cT_ref,
                 h_ref, c_ref):
    t = pl.program_id(0)

    @pl.when(t == 0)
    def _():
        h_ref[...] = jnp.zeros_like(h_ref)
        c_ref[...] = jnp.zeros_like(c_ref)

    h = h_ref[...]
    g0 = jax.lax.dot_general(h, wh0_ref[...], _DN,
                             preferred_element_type=jnp.float32)
    g1 = jax.lax.dot_general(h, wh1_ref[...], _DN,
                             preferred_element_type=jnp.float32)
    m = m_ref[0]
    gates = m * g0 + (1.0 - m) * g1 + z_ref[0]
    i = jax.nn.sigmoid(gates[:, :H])
    f = jax.nn.sigmoid(gates[:, H:2 * H])
    g = jnp.tanh(gates[:, 2 * H:3 * H])
    o = jax.nn.sigmoid(gates[:, 3 * H:])
    c = f * c_ref[...] + i * g
    h2 = o * jnp.tanh(c)
    c_ref[...] = c
    h_ref[...] = h2
    out_ref[0] = h2

    @pl.when(t == S - 1)
    def _():
        hT_ref[...] = h2
        cT_ref[...] = c


def kernel(input, input_embed, W_ih_0, W_hh_0, b_ih_0, b_hh_0,
           W_ih_1, W_hh_1, b_ih_1, b_hh_1):
    tok_sb = jnp.swapaxes(input, 0, 1)
    m_sb = (tok_sb < SPLIT).astype(jnp.float32)
    x_sb = jnp.swapaxes(input_embed, 0, 1).reshape(S * B, E)
    b0 = (b_ih_0 + b_hh_0).reshape(1, G4)
    b1 = (b_ih_1 + b_hh_1).reshape(1, G4)

    BN = 512
    NB = G4 // BN
    zsel = pl.pallas_call(
        _proj_kernel,
        grid=(NB,),
        in_specs=[
            pl.BlockSpec((S * B, E), lambda n: (0, 0)),
            pl.BlockSpec((BN, E), lambda n: (n, 0)),
            pl.BlockSpec((BN, E), lambda n: (n, 0)),
            pl.BlockSpec((1, BN), lambda n: (0, n)),
            pl.BlockSpec((1, BN), lambda n: (0, n)),
            pl.BlockSpec((S * B, 1), lambda n: (0, 0)),
        ],
        out_specs=pl.BlockSpec((S * B, BN), lambda n: (0, n)),
        out_shape=jax.ShapeDtypeStruct((S * B, G4), jnp.float32),
    )(x_sb, W_ih_0, W_ih_1, b0, b1, m_sb.reshape(S * B, 1))

    z3 = zsel.reshape(S, B, G4)

    out_sbh, hT, cT = pl.pallas_call(
        _step_kernel,
        grid=(S,),
        in_specs=[
            pl.BlockSpec((1, B, G4), lambda t: (t, 0, 0)),
            pl.BlockSpec((1, B, 1), lambda t: (t, 0, 0)),
            pl.BlockSpec((G4, H), lambda t: (0, 0)),
            pl.BlockSpec((G4, H), lambda t: (0, 0)),
        ],
        out_specs=[
            pl.BlockSpec((1, B, H), lambda t: (t, 0, 0)),
            pl.BlockSpec((B, H), lambda t: (0, 0)),
            pl.BlockSpec((B, H), lambda t: (0, 0)),
        ],
        out_shape=[
            jax.ShapeDtypeStruct((S, B, H), jnp.float32),
            jax.ShapeDtypeStruct((B, H), jnp.float32),
            jax.ShapeDtypeStruct((B, H), jnp.float32),
        ],
        scratch_shapes=[
            pltpu.VMEM((B, H), jnp.float32),
            pltpu.VMEM((B, H), jnp.float32),
        ],
    )(z3, m_sb.reshape(S, B, 1), W_hh_0, W_hh_1)

    return jnp.swapaxes(out_sbh, 0, 1), hT, cT
